# baseline (speedup 1.0000x reference)
import math

import jax
import jax.numpy as jnp
from jax.experimental import pallas as pl

_NTILE = 2


def _head(qkv_ref, e_ref, w_ref, o_ref, h):
    ch = o_ref.shape[1]
    T = o_ref.shape[2]
    scale = 1.0 / math.sqrt(math.sqrt(ch))
    e = e_ref[h]
    dn_te = (((1,), (1,)), ((), ()))
    null = jax.lax.dot_general(
        w_ref[...], e, dn_te, preferred_element_type=jnp.float32)
    qe = (qkv_ref[h, 0:ch, :] + null[0:ch, :]) * scale
    ke = (qkv_ref[h, ch:2 * ch, :] + null[ch:2 * ch, :]) * scale
    ve = qkv_ref[h, 2 * ch:3 * ch, :] + null[2 * ch:3 * ch, :]
    ve_aug = jnp.concatenate(
        [ve.T, jnp.ones((T, 8), dtype=jnp.float32)], axis=1)
    ve_aug16 = ve_aug.astype(jnp.bfloat16)
    tiles = []
    tw = T // _NTILE
    for i in range(_NTILE):
        logits = jax.lax.dot_general(
            qe[:, i * tw:(i + 1) * tw], ke, (((0,), (0,)), ((), ())),
            preferred_element_type=jnp.float32)
        ew = jnp.exp(logits).astype(jnp.bfloat16)
        tiles.append(jax.lax.dot_general(
            ew, ve_aug16, (((1,), (0,)), ((), ())),
            preferred_element_type=jnp.float32))
    a_aug = jnp.concatenate(tiles, axis=0)
    a_t = a_aug[:, 0:ch] / a_aug[:, ch:ch + 1]
    o_ref[h] = a_t.T


def _pab_kernel(qkv_ref, e_ref, w_ref, o_ref):
    _head(qkv_ref, e_ref, w_ref, o_ref, 0)
    _head(qkv_ref, e_ref, w_ref, o_ref, 1)


def kernel(qkv, cls_embedding, W_cls):
    bs, width, T = qkv.shape
    n_heads = 16
    ch = width // (3 * n_heads)
    B = bs * n_heads
    E = cls_embedding.shape[2]
    qkv_r = qkv.reshape(B, 3 * ch, T)
    out = pl.pallas_call(
        _pab_kernel,
        grid=(B // 2,),
        in_specs=[
            pl.BlockSpec((2, 3 * ch, T), lambda b: (b, 0, 0)),
            pl.BlockSpec((2, T, E), lambda b: (b, 0, 0)),
            pl.BlockSpec((3 * ch, E), lambda b: (0, 0)),
        ],
        out_specs=pl.BlockSpec((2, ch, T), lambda b: (b, 0, 0)),
        out_shape=jax.ShapeDtypeStruct((B, ch, T), qkv.dtype),
    )(qkv_r, cls_embedding, W_cls)
    return out.reshape(bs, n_heads * ch, T)
